# baseline (speedup 1.0000x reference)
import functools

import jax
import jax.numpy as jnp
import numpy as np
from jax import lax
from jax.experimental import pallas as pl
from jax.experimental.pallas import tpu as pltpu
from jax.experimental.pallas import tpu_sc as plsc

_B = 4096
_S = 50
_D = 64

_NC = 2
_NS = 16
_NW = _NC * _NS
_L = 16

_ROWS = _B * _S
_RPW = _ROWS // _NW
_CHUNK = 80
_NCHUNK = _RPW // _CHUNK
_POSREP = 400


def _pos_rep_body(radius_ref, period_ref, phase_ref, out_ref):
    rows_i = lax.broadcasted_iota(jnp.int32, (_POSREP, 1), 0)
    pos = (rows_i % _S).astype(jnp.float32)
    out_ref[...] = radius_ref[...] * jnp.cos(
        2.0 * np.pi * pos / period_ref[...] + phase_ref[...]
    )


def _pos_bcast_body(radius_ref, period_ref, phase_ref, out_ref):
    pos = lax.broadcasted_iota(jnp.int32, (1, _S, 1), 1).astype(jnp.float32)
    pe = radius_ref[...] * jnp.cos(
        2.0 * np.pi * pos / period_ref[...] + phase_ref[...]
    )
    out_ref[...] = jnp.broadcast_to(pe, out_ref.shape)


_BB = 128


def _sc_gather_add(idx_hbm, table_hbm, posrep_hbm, x_hbm, idx_v, pos_v, buf_v, sem):
    wid = lax.axis_index("s") * _NC + lax.axis_index("c")
    base = wid * _RPW
    pltpu.sync_copy(idx_hbm.at[pl.ds(wid * _NCHUNK, _NCHUNK)], idx_v)
    pltpu.sync_copy(posrep_hbm, pos_v)

    def chunk_body(ch, carry):
        pltpu.async_copy(table_hbm.at[idx_v.at[ch]], buf_v, sem).wait()
        poff = lax.rem(ch * _CHUNK, _POSREP)

        def row_body(r, c2):
            pr = poff + r
            for c4 in range(_D // _L):
                sl = pl.ds(c4 * _L, _L)
                buf_v[r, sl] = buf_v[r, sl] + pos_v[pr, sl]
            return c2

        lax.fori_loop(0, _CHUNK, row_body, carry)
        pltpu.sync_copy(buf_v, x_hbm.at[pl.ds(base + ch * _CHUNK, _CHUNK)])
        return carry

    lax.fori_loop(0, _NCHUNK, chunk_body, 0)


@functools.cache
def _build_gather_add():
    sc_mesh = plsc.VectorSubcoreMesh(
        core_axis_name="c", subcore_axis_name="s", num_cores=_NC, num_subcores=_NS
    )
    return pl.kernel(
        _sc_gather_add,
        out_type=jax.ShapeDtypeStruct((_ROWS, _D), jnp.float32),
        mesh=sc_mesh,
        scratch_types=[
            pltpu.VMEM((_NCHUNK, _CHUNK), jnp.int32),
            pltpu.VMEM((_POSREP, _D), jnp.float32),
            pltpu.VMEM((_CHUNK, _D), jnp.float32),
            pltpu.SemaphoreType.DMA,
        ],
        compiler_params=pltpu.CompilerParams(use_tc_tiling_on_sc=False),
    )


_pos_rep = pl.pallas_call(
    _pos_rep_body,
    out_shape=jax.ShapeDtypeStruct((_POSREP, _D), jnp.float32),
)

_pos_bcast = pl.pallas_call(
    _pos_bcast_body,
    grid=(_B // _BB,),
    in_specs=[
        pl.BlockSpec((1, 1, _D), lambda i: (0, 0, 0)),
        pl.BlockSpec((1, 1, _D), lambda i: (0, 0, 0)),
        pl.BlockSpec((1, 1, _D), lambda i: (0, 0, 0)),
    ],
    out_specs=pl.BlockSpec((_BB, _S, _D), lambda i: (i, 0, 0)),
    out_shape=jax.ShapeDtypeStruct((_B, _S, _D), jnp.float32),
)


def kernel(sequence, token_table, init_radius, period, init_phase):
    seq_flat = sequence.reshape(-1).astype(jnp.int32)
    idx2d = seq_flat.reshape(_NW * _NCHUNK, _CHUNK)
    r2 = init_radius.reshape(1, _D)
    p2 = period.reshape(1, _D)
    f2 = init_phase.reshape(1, _D)
    posrep = _pos_rep(r2, p2, f2)
    x_flat = _build_gather_add()(idx2d, token_table, posrep)
    position_e = _pos_bcast(
        init_radius.reshape(1, 1, _D),
        period.reshape(1, 1, _D),
        init_phase.reshape(1, 1, _D),
    )
    x = x_flat.reshape(_B, _S, _D)
    return (x, init_radius, period, init_phase, position_e)


# speedup vs baseline: 1.1641x; 1.1641x over previous
import functools

import jax
import jax.numpy as jnp
import numpy as np
from jax import lax
from jax.experimental import pallas as pl
from jax.experimental.pallas import tpu as pltpu
from jax.experimental.pallas import tpu_sc as plsc

_B = 4096
_S = 50
_D = 64

_NC = 2
_NS = 16
_NW = _NC * _NS
_L = 16

_SPW = _B // _NW


def _pos_rep_body(radius_ref, period_ref, phase_ref, out_ref):
    rows_i = lax.broadcasted_iota(jnp.int32, (_S, 1), 0)
    pos = rows_i.astype(jnp.float32)
    out_ref[...] = radius_ref[...] * jnp.cos(
        2.0 * np.pi * pos / period_ref[...] + phase_ref[...]
    )


def _pos_bcast_body(radius_ref, period_ref, phase_ref, out_ref):
    pos = lax.broadcasted_iota(jnp.int32, (1, _S, 1), 1).astype(jnp.float32)
    pe = radius_ref[...] * jnp.cos(
        2.0 * np.pi * pos / period_ref[...] + phase_ref[...]
    )
    out_ref[...] = jnp.broadcast_to(pe, out_ref.shape)


_BB = 128


def _seq_add(dst_ref, src_ref, pos_ref):
    def row_body(r, c):
        for c4 in range(_D // _L):
            sl = pl.ds(c4 * _L, _L)
            dst_ref[r, sl] = src_ref[r, sl] + pos_ref[r, sl]
        return c

    lax.fori_loop(0, _S, row_body, 0)


def _sc_gather_add(
    seq_hbm, table_hbm, pos_hbm, x_hbm,
    idx_v, pos_v, r0, r1, w0, w1, gs0, gs1, ws0, ws1,
):
    wid = lax.axis_index("s") * _NC + lax.axis_index("c")
    sbase = wid * _SPW
    pltpu.sync_copy(seq_hbm.at[pl.ds(sbase, _SPW)], idx_v)
    pltpu.sync_copy(pos_hbm, pos_v)

    pltpu.async_copy(table_hbm.at[idx_v.at[0]], r0, gs0)
    pltpu.async_copy(table_hbm.at[idx_v.at[1]], r1, gs1)

    def body(g, carry):
        for slot, (rb, wb, gs, ws) in enumerate(
            ((r0, w0, gs0, ws0), (r1, w1, gs1, ws1))
        ):
            si = 2 * g + slot
            pltpu.make_async_copy(table_hbm.at[idx_v.at[si]], rb, gs).wait()
            _seq_add(wb, rb, pos_v)
            @pl.when(si + 2 < 2 * _SPW // 2)
            def _():
                pltpu.async_copy(table_hbm.at[idx_v.at[si + 2]], rb, gs)
            @pl.when(si >= 2)
            def _():
                pltpu.make_async_copy(wb, x_hbm.at[sbase + si], ws).wait()
            pltpu.async_copy(wb, x_hbm.at[sbase + si], ws)
        return carry

    lax.fori_loop(0, _SPW // 2, body, 0)
    pltpu.make_async_copy(w0, x_hbm.at[sbase], ws0).wait()
    pltpu.make_async_copy(w1, x_hbm.at[sbase], ws1).wait()


@functools.cache
def _build_gather_add():
    sc_mesh = plsc.VectorSubcoreMesh(
        core_axis_name="c", subcore_axis_name="s", num_cores=_NC, num_subcores=_NS
    )
    return pl.kernel(
        _sc_gather_add,
        out_type=jax.ShapeDtypeStruct((_B, _S, _D), jnp.float32),
        mesh=sc_mesh,
        scratch_types=[
            pltpu.VMEM((_SPW, _S), jnp.int32),
            pltpu.VMEM((_S, _D), jnp.float32),
            pltpu.VMEM((_S, _D), jnp.float32),
            pltpu.VMEM((_S, _D), jnp.float32),
            pltpu.VMEM((_S, _D), jnp.float32),
            pltpu.VMEM((_S, _D), jnp.float32),
            pltpu.SemaphoreType.DMA,
            pltpu.SemaphoreType.DMA,
            pltpu.SemaphoreType.DMA,
            pltpu.SemaphoreType.DMA,
        ],
        compiler_params=pltpu.CompilerParams(use_tc_tiling_on_sc=False),
    )


_pos_rep = pl.pallas_call(
    _pos_rep_body,
    out_shape=jax.ShapeDtypeStruct((_S, _D), jnp.float32),
)

_pos_bcast = pl.pallas_call(
    _pos_bcast_body,
    grid=(_B // _BB,),
    in_specs=[
        pl.BlockSpec((1, 1, _D), lambda i: (0, 0, 0)),
        pl.BlockSpec((1, 1, _D), lambda i: (0, 0, 0)),
        pl.BlockSpec((1, 1, _D), lambda i: (0, 0, 0)),
    ],
    out_specs=pl.BlockSpec((_BB, _S, _D), lambda i: (i, 0, 0)),
    out_shape=jax.ShapeDtypeStruct((_B, _S, _D), jnp.float32),
)


def kernel(sequence, token_table, init_radius, period, init_phase):
    seq = sequence.astype(jnp.int32)
    r2 = init_radius.reshape(1, _D)
    p2 = period.reshape(1, _D)
    f2 = init_phase.reshape(1, _D)
    pos = _pos_rep(r2, p2, f2)
    x = _build_gather_add()(seq, token_table, pos)
    position_e = _pos_bcast(
        init_radius.reshape(1, 1, _D),
        period.reshape(1, 1, _D),
        init_phase.reshape(1, 1, _D),
    )
    return (x, init_radius, period, init_phase, position_e)


# speedup vs baseline: 1.2370x; 1.0626x over previous
import functools

import jax
import jax.numpy as jnp
import numpy as np
from jax import lax
from jax.experimental import pallas as pl
from jax.experimental.pallas import tpu as pltpu
from jax.experimental.pallas import tpu_sc as plsc

_B = 4096
_S = 50
_D = 64

_NC = 2
_NS = 16
_NW = _NC * _NS
_L = 16

_SPW = _B // _NW
_VP = 500000


def _pos_rep_body(radius_ref, period_ref, phase_ref, out_ref):
    rows_i = lax.broadcasted_iota(jnp.int32, (64, 1), 0)
    pos = rows_i.astype(jnp.float32)
    out_ref[...] = radius_ref[...] * jnp.cos(
        2.0 * np.pi * pos / period_ref[...] + phase_ref[...]
    )


def _pos_bcast_body(radius_ref, period_ref, phase_ref, out_ref):
    pos = lax.broadcasted_iota(jnp.int32, (1, _S, 1), 1).astype(jnp.float32)
    pe = radius_ref[...] * jnp.cos(
        2.0 * np.pi * pos / period_ref[...] + phase_ref[...]
    )
    out_ref[...] = jnp.broadcast_to(pe, out_ref.shape)


_BB = 128


def _seq_combine(dst_ref, src_ref, par_ref, si, pos_ref):
    def row_body(r, c):
        h = par_ref[si, pl.ds(r, _L)][0]
        hf = jnp.full((_L,), h, jnp.int32).astype(jnp.float32)
        for c4 in range(_D // _L):
            sl = pl.ds(c4 * _L, _L)
            sh = pl.ds(_D + c4 * _L, _L)
            lo = src_ref[r, sl]
            hi = src_ref[r, sh]
            dst_ref[r, sl] = lo + hf * (hi - lo) + pos_ref[r, sl]
        return c

    lax.fori_loop(0, _S, row_body, 0)


def _sc_gather_add(
    pairs_hbm, par_hbm, table_hbm, pos_hbm, x_hbm,
    idx_v, par_v, pos_v, r0, r1, w0, w1, gs0, gs1, ws0, ws1,
):
    wid = lax.axis_index("s") * _NC + lax.axis_index("c")
    sbase = wid * _SPW
    pltpu.sync_copy(pairs_hbm.at[pl.ds(sbase, _SPW)], idx_v)
    pltpu.sync_copy(par_hbm.at[pl.ds(sbase, _SPW)], par_v)
    pltpu.sync_copy(pos_hbm, pos_v)

    pltpu.async_copy(table_hbm.at[idx_v.at[0, pl.ds(0, _S)]], r0.at[pl.ds(0, _S)], gs0)
    pltpu.async_copy(table_hbm.at[idx_v.at[1, pl.ds(0, _S)]], r1.at[pl.ds(0, _S)], gs1)

    def body(g, carry):
        for slot, (rb, wb, gs, ws) in enumerate(
            ((r0, w0, gs0, ws0), (r1, w1, gs1, ws1))
        ):
            si = 2 * g + slot
            pltpu.make_async_copy(
                table_hbm.at[idx_v.at[si, pl.ds(0, _S)]], rb.at[pl.ds(0, _S)], gs
            ).wait()
            _seq_combine(wb, rb, par_v, si, pos_v)
            @pl.when(si + 2 < _SPW)
            def _():
                pltpu.async_copy(
                    table_hbm.at[idx_v.at[si + 2, pl.ds(0, _S)]],
                    rb.at[pl.ds(0, _S)], gs,
                )
            @pl.when(si >= 2)
            def _():
                pltpu.make_async_copy(
                    wb.at[pl.ds(0, _S)], x_hbm.at[sbase + si], ws
                ).wait()
            pltpu.async_copy(wb.at[pl.ds(0, _S)], x_hbm.at[sbase + si], ws)
        return carry

    lax.fori_loop(0, _SPW // 2, body, 0)
    pltpu.make_async_copy(w0.at[pl.ds(0, _S)], x_hbm.at[sbase], ws0).wait()
    pltpu.make_async_copy(w1.at[pl.ds(0, _S)], x_hbm.at[sbase], ws1).wait()


@functools.cache
def _build_gather_add():
    sc_mesh = plsc.VectorSubcoreMesh(
        core_axis_name="c", subcore_axis_name="s", num_cores=_NC, num_subcores=_NS
    )
    return pl.kernel(
        _sc_gather_add,
        out_type=jax.ShapeDtypeStruct((_B, _S, _D), jnp.float32),
        mesh=sc_mesh,
        scratch_types=[
            pltpu.VMEM((_SPW, 128), jnp.int32),
            pltpu.VMEM((_SPW, 128), jnp.int32),
            pltpu.VMEM((64, 128), jnp.float32),
            pltpu.VMEM((56, 128), jnp.float32),
            pltpu.VMEM((56, 128), jnp.float32),
            pltpu.VMEM((56, _D), jnp.float32),
            pltpu.VMEM((56, _D), jnp.float32),
            pltpu.SemaphoreType.DMA,
            pltpu.SemaphoreType.DMA,
            pltpu.SemaphoreType.DMA,
            pltpu.SemaphoreType.DMA,
        ],
        compiler_params=pltpu.CompilerParams(use_tc_tiling_on_sc=True),
    )


_pos_rep = pl.pallas_call(
    _pos_rep_body,
    out_shape=jax.ShapeDtypeStruct((64, 128), jnp.float32),
)

_pos_bcast = pl.pallas_call(
    _pos_bcast_body,
    grid=(_B // _BB,),
    in_specs=[
        pl.BlockSpec((1, 1, _D), lambda i: (0, 0, 0)),
        pl.BlockSpec((1, 1, _D), lambda i: (0, 0, 0)),
        pl.BlockSpec((1, 1, _D), lambda i: (0, 0, 0)),
    ],
    out_specs=pl.BlockSpec((_BB, _S, _D), lambda i: (i, 0, 0)),
    out_shape=jax.ShapeDtypeStruct((_B, _S, _D), jnp.float32),
)


def kernel(sequence, token_table, init_radius, period, init_phase):
    seq = sequence.astype(jnp.int32)
    pairs = jnp.pad(seq >> 1, ((0, 0), (0, 128 - _S)))
    par = jnp.pad(seq & 1, ((0, 0), (0, 128 - _S)))
    table2 = token_table.reshape(_VP, 128)
    r2 = jnp.pad(init_radius.reshape(1, _D), ((0, 0), (0, 64)))
    p2 = jnp.pad(period.reshape(1, _D), ((0, 0), (0, 64)), constant_values=1.0)
    f2 = jnp.pad(init_phase.reshape(1, _D), ((0, 0), (0, 64)))
    pos = _pos_rep(r2, p2, f2)
    x = _build_gather_add()(pairs, par, table2, pos)
    position_e = _pos_bcast(
        init_radius.reshape(1, 1, _D),
        period.reshape(1, 1, _D),
        init_phase.reshape(1, 1, _D),
    )
    return (x, init_radius, period, init_phase, position_e)
